# baseline (speedup 1.0000x reference)
import jax
import jax.numpy as jnp
from jax.experimental import pallas as pl

H = 4
S = 128
SENT = 8


def _combine_body(logits_ref, dist_ref, out_ref):
    logits = logits_ref[...]
    dist = dist_ref[...]
    x = 1.0 - dist
    m = jnp.maximum(jnp.max(x, axis=2, keepdims=True), 0.0)
    e = jnp.exp(x - m)
    denom = jnp.sum(e, axis=2) + SENT * jnp.exp(-m[:, :, 0])
    num = jnp.sum(logits * e, axis=2)
    out_ref[...] = jnp.mean(num / denom, axis=0)


def kernel(edge, pos, field, uncertainty, edge_mat, mid0, mid1):
    src = edge[:, 0]
    dst = edge[:, 1]
    n = edge.shape[0]
    hidx = jnp.arange(H)[:, None, None]
    srcdiff = pos[:, src][:, :, None, :] - pos[hidx, mid0]
    mem1 = uncertainty * edge_mat[mid0, dst[None, :, None]]
    logits1 = jnp.einsum('hnsd,hnd->hns', srcdiff, field[:, dst]) + mem1
    dstdiff = pos[:, dst][:, :, None, :] - pos[hidx, mid1]
    mem2 = uncertainty * edge_mat[src[None, :, None], mid1]
    logits2 = jnp.einsum('hnsd,hnd->hns', dstdiff, field[:, src]) + mem2
    logits = jnp.concatenate([logits1, logits2], axis=2)
    dist = jnp.linalg.norm(jnp.concatenate([srcdiff, dstdiff], axis=2), axis=3)
    return pl.pallas_call(
        _combine_body,
        out_shape=jax.ShapeDtypeStruct((n,), jnp.float32),
    )(logits, dist)


# speedup vs baseline: 9.2798x; 9.2798x over previous
import functools

import jax
import jax.numpy as jnp
from jax import lax
from jax.experimental import pallas as pl
from jax.experimental.pallas import tpu as pltpu
from jax.experimental.pallas import tpu_sc as plsc

H = 4
NE = 1024
S = 128
D = 64
N_NODES = 10000
SENT = 8

_info = plsc.get_sparse_core_info()
_NC, _NS, _L = _info.num_cores, _info.num_subcores, _info.num_lanes
_NW = _NC * _NS
_EPW = NE // _NW


def _i32(x):
    return jnp.full((_L,), x, dtype=jnp.int32)


def _f32(x):
    return jnp.full((_L,), x, dtype=jnp.float32)


def _rsqrt(x):
    i = plsc.bitcast(x, jnp.int32)
    i = jnp.int32(0x5F3759DF) - lax.shift_right_logical(i, 1)
    y = plsc.bitcast(i, jnp.float32)
    for _ in range(3):
        y = y * (1.5 - 0.5 * x * y * y)
    return y


def _make_sc_kernel():
    mesh = plsc.VectorSubcoreMesh(core_axis_name="c", subcore_axis_name="s")

    @functools.partial(
        pl.kernel,
        mesh=mesh,
        out_type=jax.ShapeDtypeStruct((NE,), jnp.float32),
        compiler_params=pltpu.CompilerParams(needs_layout_passes=False,
                                             use_tc_tiling_on_sc=False),
        scratch_types=[
            pltpu.VMEM((2 * _EPW,), jnp.int32),
            pltpu.VMEM((2 * _EPW,), jnp.int32),
            pltpu.VMEM((H, 2 * _EPW, D), jnp.float32),
            pltpu.VMEM((H, 2 * _EPW, D), jnp.float32),
            pltpu.VMEM((S,), jnp.int32),
            pltpu.VMEM((S,), jnp.int32),
            pltpu.VMEM((S, D), jnp.float32),
            pltpu.VMEM((S, D), jnp.float32),
            pltpu.VMEM((S,), jnp.int32),
            pltpu.VMEM((S,), jnp.int32),
            pltpu.VMEM((S,), jnp.float32),
            pltpu.VMEM((S,), jnp.float32),
            pltpu.VMEM((_L,), jnp.float32),
            pltpu.VMEM((_EPW,), jnp.float32),
            pltpu.SemaphoreType.DMA,
            pltpu.SemaphoreType.DMA,
            pltpu.SemaphoreType.DMA,
            pltpu.SemaphoreType.DMA,
        ],
    )
    def sc_kernel(edge_hbm, pos_hbm, field_hbm, unc_hbm, emat_hbm,
                  mid0_hbm, mid1_hbm, out_hbm,
                  edge_v, sd_idx, rows_pos, rows_field,
                  g0idx, g1idx, g0rows, g1rows,
                  em0, em1, m0v, m1v, uncv, out_acc,
                  semA, semB, semC, semD):
        wid = lax.axis_index("s") * _NC + lax.axis_index("c")
        base = wid * _EPW

        iota = jnp.arange(_L, dtype=jnp.int32)
        row_idx = [g * _L + iota for g in range(8)]
        zeros = _i32(0)
        ones = _i32(1)

        pltpu.sync_copy(unc_hbm, uncv)
        pltpu.sync_copy(edge_hbm.at[pl.ds(2 * base, 2 * _EPW)], edge_v)

        for half in range(2):
            for g in range(2):
                v = plsc.load_gather(edge_v, [(g * _L + iota) * 2 + half])
                sd_idx[pl.ds(half * _EPW + g * _L, _L)] = v

        cps = []
        for h in range(H):
            cps.append(pltpu.async_copy(pos_hbm.at[h].at[sd_idx],
                                        rows_pos.at[h], semA))
            cps.append(pltpu.async_copy(field_hbm.at[h].at[sd_idx],
                                        rows_field.at[h], semB))
        for cp in cps:
            cp.wait()

        u_vec = uncv[...]

        def edge_body(jj, carry):
            src_b = plsc.load_gather(edge_v, [_i32(2 * jj)])
            dst_b = plsc.load_gather(edge_v, [_i32(2 * jj + 1)])

            hsum = jnp.zeros((_L,), jnp.float32)
            for h in range(H):
                pltpu.sync_copy(mid0_hbm.at[h, base + jj, :], g0idx)
                pltpu.sync_copy(mid1_hbm.at[h, base + jj, :], g1idx)
                cpg0 = pltpu.async_copy(pos_hbm.at[h].at[g0idx], g0rows, semA)
                cpg1 = pltpu.async_copy(pos_hbm.at[h].at[g1idx], g1rows, semB)

                for g in range(8):
                    mv0 = g0idx[pl.ds(g * _L, _L)]
                    mv1 = g1idx[pl.ds(g * _L, _L)]
                    em0[pl.ds(g * _L, _L)] = mv0 * N_NODES + dst_b
                    em1[pl.ds(g * _L, _L)] = src_b * N_NODES + mv1
                cpm0 = pltpu.async_copy(emat_hbm.at[em0], m0v, semC)
                cpm1 = pltpu.async_copy(emat_hbm.at[em1], m1v, semD)

                logit_vecs = []
                dist_vecs = []
                for half in range(2):
                    grows = g0rows if half == 0 else g1rows
                    p_row = _i32(jj + (half * _EPW))
                    f_row = _i32(jj + ((1 - half) * _EPW))
                    h_b = _i32(h)
                    (cpg0 if half == 0 else cpg1).wait()

                    def d_body(d, accs, grows=grows, p_row=p_row,
                               f_row=f_row, h_b=h_b):
                        dcol = _i32(d)
                        ps = plsc.load_gather(rows_pos, [h_b, p_row, dcol])
                        fd = plsc.load_gather(rows_field, [h_b, f_row, dcol])
                        out = []
                        for g in range(8):
                            v = plsc.load_gather(grows, [row_idx[g], dcol])
                            df = ps - v
                            out.append(accs[2 * g] + df * fd)
                            out.append(accs[2 * g + 1] + df * df)
                        return tuple(out)

                    accs = lax.fori_loop(
                        0, D, d_body,
                        tuple(jnp.zeros((_L,), jnp.float32) for _ in range(16)))

                    (cpm0 if half == 0 else cpm1).wait()
                    mref = m0v if half == 0 else m1v
                    for g in range(8):
                        al, ad = accs[2 * g], accs[2 * g + 1]
                        logit_vecs.append(al + u_vec * mref[pl.ds(g * _L, _L)])
                        x = jnp.maximum(ad, jnp.float32(1e-30))
                        dist_vecs.append(ad * _rsqrt(x))

                dmin = dist_vecs[0]
                for k in range(1, 16):
                    dmin = jnp.minimum(dmin, dist_vecs[k])
                m = jnp.maximum(1.0 - jnp.min(dmin, axis=0), jnp.float32(0.0))
                num = jnp.zeros((_L,), jnp.float32)
                den = jnp.zeros((_L,), jnp.float32)
                for k in range(16):
                    e = jnp.exp((1.0 - dist_vecs[k]) - m)
                    num = num + logit_vecs[k] * e
                    den = den + e
                den = den + jnp.exp(_f32(0.0) - m) * jnp.float32(SENT / _L)
                num_s = _f32(0.0) + jnp.sum(num, axis=0)
                den_s = _f32(0.0) + jnp.sum(den, axis=0)
                hsum = hsum + num_s / den_s

            plsc.store_scatter(out_acc, [_i32(jj)],
                               hsum * jnp.float32(1.0 / H),
                               mask=iota == 0)
            return carry

        lax.fori_loop(0, _EPW, edge_body, jnp.int32(0))

        pltpu.sync_copy(out_acc, out_hbm.at[pl.ds(base, _EPW)])

    return sc_kernel


_SC_KERNEL = _make_sc_kernel()


def kernel(edge, pos, field, uncertainty, edge_mat, mid0, mid1):
    unc16 = jnp.broadcast_to(uncertainty.reshape(1), (_L,)).astype(jnp.float32)
    emat_flat = edge_mat.reshape(N_NODES * N_NODES)
    edge_flat = edge.reshape(2 * NE)
    return _SC_KERNEL(edge_flat, pos, field, unc16, emat_flat, mid0, mid1)


# speedup vs baseline: 10.0860x; 1.0869x over previous
import functools

import jax
import jax.numpy as jnp
from jax import lax
from jax.experimental import pallas as pl
from jax.experimental.pallas import tpu as pltpu
from jax.experimental.pallas import tpu_sc as plsc

H = 4
NE = 1024
S = 128
D = 64
N_NODES = 10000
SENT = 8

_info = plsc.get_sparse_core_info()
_NC, _NS, _L = _info.num_cores, _info.num_subcores, _info.num_lanes
_NW = _NC * _NS
_EPW = NE // _NW
_PAIRS = H * _EPW


def _i32(x):
    return jnp.full((_L,), x, dtype=jnp.int32)


def _f32(x):
    return jnp.full((_L,), x, dtype=jnp.float32)


def _rsqrt(x):
    i = plsc.bitcast(x, jnp.int32)
    i = jnp.int32(0x5F3759DF) - lax.shift_right_logical(i, 1)
    y = plsc.bitcast(i, jnp.float32)
    for _ in range(3):
        y = y * (1.5 - 0.5 * x * y * y)
    return y


def _make_sc_kernel():
    mesh = plsc.VectorSubcoreMesh(core_axis_name="c", subcore_axis_name="s")

    @functools.partial(
        pl.kernel,
        mesh=mesh,
        out_type=jax.ShapeDtypeStruct((NE,), jnp.float32),
        compiler_params=pltpu.CompilerParams(needs_layout_passes=False,
                                             use_tc_tiling_on_sc=False),
        scratch_types=[
            pltpu.VMEM((2 * _EPW,), jnp.int32),
            pltpu.VMEM((2 * _EPW,), jnp.int32),
            pltpu.VMEM((H, 2 * _EPW, D), jnp.float32),
            pltpu.VMEM((H, 2 * _EPW, D), jnp.float32),
            pltpu.VMEM((H, _EPW, S), jnp.int32),
            pltpu.VMEM((H, _EPW, S), jnp.int32),
            pltpu.VMEM((2, S, D), jnp.float32),
            pltpu.VMEM((2, S, D), jnp.float32),
            pltpu.VMEM((2, S), jnp.int32),
            pltpu.VMEM((2, S), jnp.int32),
            pltpu.VMEM((2, S), jnp.float32),
            pltpu.VMEM((2, S), jnp.float32),
            pltpu.VMEM((2 * S,), jnp.float32),
            pltpu.VMEM((2 * S,), jnp.float32),
            pltpu.VMEM((_L,), jnp.float32),
            pltpu.VMEM((_EPW,), jnp.float32),
            [[pltpu.SemaphoreType.DMA] * 4] * 2,
        ],
    )
    def sc_kernel(edge_hbm, pos_hbm, field_hbm, unc_hbm, emat_hbm,
                  mid0_hbm, mid1_hbm, out_hbm,
                  edge_v, sd_idx, rows_pos, rows_field,
                  mid0_v, mid1_v, g0rows, g1rows,
                  em0, em1, m0v, m1v, logit_v, dist_v, uncv, out_acc,
                  sems):
        wid = lax.axis_index("s") * _NC + lax.axis_index("c")
        base = wid * _EPW

        iota = jnp.arange(_L, dtype=jnp.int32)
        row_idx = [g * _L + iota for g in range(8)]
        zeros16 = jnp.zeros((_L,), jnp.float32)

        pltpu.sync_copy(unc_hbm, uncv)
        pltpu.sync_copy(edge_hbm.at[pl.ds(2 * base, 2 * _EPW)], edge_v)
        for h in range(H):
            pltpu.sync_copy(mid0_hbm.at[h, pl.ds(base, _EPW), :], mid0_v.at[h])
            pltpu.sync_copy(mid1_hbm.at[h, pl.ds(base, _EPW), :], mid1_v.at[h])

        for half in range(2):
            for g in range(2):
                v = plsc.load_gather(edge_v, [(g * _L + iota) * 2 + half])
                sd_idx[pl.ds(half * _EPW + g * _L, _L)] = v

        cps = []
        for h in range(H):
            cps.append(pltpu.async_copy(pos_hbm.at[h].at[sd_idx],
                                        rows_pos.at[h], sems[0][0]))
            cps.append(pltpu.async_copy(field_hbm.at[h].at[sd_idx],
                                        rows_field.at[h], sems[0][1]))
        for cp in cps:
            cp.wait()

        out_acc[pl.ds(0, _L)] = zeros16
        out_acc[pl.ds(_L, _L)] = zeros16
        u_vec = uncv[...]

        def issue(p, b):
            h = lax.shift_right_logical(p, 5)
            jj = lax.bitwise_and(p, _EPW - 1)
            h_b = _i32(h)
            jj_b = _i32(jj)
            pltpu.async_copy(pos_hbm.at[h].at[mid0_v.at[h, jj]],
                             g0rows.at[b], sems[b][0])
            pltpu.async_copy(pos_hbm.at[h].at[mid1_v.at[h, jj]],
                             g1rows.at[b], sems[b][1])
            src_b = plsc.load_gather(edge_v, [jj_b * 2])
            dst_b = plsc.load_gather(edge_v, [jj_b * 2 + 1])
            bb = _i32(b)
            for g in range(8):
                mv0 = plsc.load_gather(mid0_v, [h_b, jj_b, row_idx[g]])
                mv1 = plsc.load_gather(mid1_v, [h_b, jj_b, row_idx[g]])
                plsc.store_scatter(em0, [bb, row_idx[g]],
                                   mv0 * N_NODES + dst_b)
                plsc.store_scatter(em1, [bb, row_idx[g]],
                                   src_b * N_NODES + mv1)
            pltpu.async_copy(emat_hbm.at[em0.at[b]], m0v.at[b], sems[b][2])
            pltpu.async_copy(emat_hbm.at[em1.at[b]], m1v.at[b], sems[b][3])

        def wait_pair(b):
            pltpu.make_async_copy(pos_hbm.at[0].at[mid0_v.at[0, 0]],
                                  g0rows.at[b], sems[b][0]).wait()
            pltpu.make_async_copy(pos_hbm.at[0].at[mid0_v.at[0, 0]],
                                  g1rows.at[b], sems[b][1]).wait()
            pltpu.make_async_copy(emat_hbm.at[em0.at[b]],
                                  m0v.at[b], sems[b][2]).wait()
            pltpu.make_async_copy(emat_hbm.at[em0.at[b]],
                                  m1v.at[b], sems[b][3]).wait()

        def compute(p, b):
            h = lax.shift_right_logical(p, 5)
            jj = lax.bitwise_and(p, _EPW - 1)
            h_b = _i32(h)
            bb = _i32(b)

            for half in range(2):
                grows = g0rows if half == 0 else g1rows
                p_row = _i32(jj + (half * _EPW))
                f_row = _i32(jj + ((1 - half) * _EPW))

                def d_body(d, accs, grows=grows, p_row=p_row, f_row=f_row):
                    dcol = _i32(d)
                    ps = plsc.load_gather(rows_pos, [h_b, p_row, dcol])
                    fd = plsc.load_gather(rows_field, [h_b, f_row, dcol])
                    out = []
                    for g in range(8):
                        v = plsc.load_gather(grows, [bb, row_idx[g], dcol])
                        df = ps - v
                        out.append(accs[2 * g] + df * fd)
                        out.append(accs[2 * g + 1] + df * df)
                    return tuple(out)

                accs = plsc.parallel_loop(
                    0, D, unroll=4,
                    carry=tuple(zeros16 for _ in range(16)))(d_body)

                mref = m0v if half == 0 else m1v
                for g in range(8):
                    al, ad = accs[2 * g], accs[2 * g + 1]
                    mem = plsc.load_gather(mref, [bb, row_idx[g]])
                    off = half * S + g * _L
                    logit_v[pl.ds(off, _L)] = al + u_vec * mem
                    x = jnp.maximum(ad, jnp.float32(1e-30))
                    dist_v[pl.ds(off, _L)] = ad * _rsqrt(x)

            dmin = dist_v[pl.ds(0, _L)]
            for k in range(1, 16):
                dmin = jnp.minimum(dmin, dist_v[pl.ds(k * _L, _L)])
            m = jnp.maximum(1.0 - jnp.min(dmin, axis=0), jnp.float32(0.0))
            num = zeros16
            den = zeros16
            for k in range(16):
                e = jnp.exp((1.0 - dist_v[pl.ds(k * _L, _L)]) - m)
                num = num + logit_v[pl.ds(k * _L, _L)] * e
                den = den + e
            den = den + jnp.exp(_f32(0.0) - m) * jnp.float32(SENT / _L)
            num_s = _f32(0.0) + jnp.sum(num, axis=0)
            den_s = _f32(0.0) + jnp.sum(den, axis=0)
            val = num_s / den_s

            jj_b = _i32(jj)
            cur = plsc.load_gather(out_acc, [jj_b])
            plsc.store_scatter(out_acc, [jj_b],
                               cur + val * jnp.float32(1.0 / H),
                               mask=iota == 0)

        issue(jnp.int32(0), 0)

        def pair_body(i, carry):
            p0 = 2 * i
            issue(p0 + 1, 1)
            wait_pair(0)
            compute(p0, 0)

            @pl.when(i < _PAIRS // 2 - 1)
            def _():
                issue(p0 + 2, 0)

            wait_pair(1)
            compute(p0 + 1, 1)
            return carry

        lax.fori_loop(0, _PAIRS // 2, pair_body, jnp.int32(0))

        pltpu.sync_copy(out_acc, out_hbm.at[pl.ds(base, _EPW)])

    return sc_kernel


_SC_KERNEL = _make_sc_kernel()


def kernel(edge, pos, field, uncertainty, edge_mat, mid0, mid1):
    unc16 = jnp.broadcast_to(uncertainty.reshape(1), (_L,)).astype(jnp.float32)
    emat_flat = edge_mat.reshape(N_NODES * N_NODES)
    edge_flat = edge.reshape(2 * NE)
    return _SC_KERNEL(edge_flat, pos, field, unc16, emat_flat, mid0, mid1)


# speedup vs baseline: 26.8074x; 2.6579x over previous
import functools

import jax
import jax.numpy as jnp
from jax import lax
from jax.experimental import pallas as pl
from jax.experimental.pallas import tpu as pltpu
from jax.experimental.pallas import tpu_sc as plsc

H = 4
NE = 1024
S = 128
D = 64
N_NODES = 10000
SENT = 8

_info = plsc.get_sparse_core_info()
_NC, _NS, _L = _info.num_cores, _info.num_subcores, _info.num_lanes
_NW = _NC * _NS
_EPW = NE // _NW
_PAIRS = H * _EPW


def _i32(x):
    return jnp.full((_L,), x, dtype=jnp.int32)


def _f32(x):
    return jnp.full((_L,), x, dtype=jnp.float32)


def _rsqrt(x):
    i = plsc.bitcast(x, jnp.int32)
    i = jnp.int32(0x5F3759DF) - lax.shift_right_logical(i, 1)
    y = plsc.bitcast(i, jnp.float32)
    for _ in range(3):
        y = y * (1.5 - 0.5 * x * y * y)
    return y


def _make_sc_kernel():
    mesh = plsc.VectorSubcoreMesh(core_axis_name="c", subcore_axis_name="s")

    @functools.partial(
        pl.kernel,
        mesh=mesh,
        out_type=jax.ShapeDtypeStruct((NE,), jnp.float32),
        compiler_params=pltpu.CompilerParams(needs_layout_passes=False,
                                             use_tc_tiling_on_sc=False),
        scratch_types=[
            pltpu.VMEM((2 * _EPW,), jnp.int32),
            pltpu.VMEM((2 * _EPW,), jnp.int32),
            pltpu.VMEM((H, 2 * _EPW, D), jnp.float32),
            pltpu.VMEM((H, 2 * _EPW, D), jnp.float32),
            pltpu.VMEM((H, _EPW, S), jnp.int32),
            pltpu.VMEM((H, _EPW, S), jnp.int32),
            pltpu.VMEM((2, S, D), jnp.float32),
            pltpu.VMEM((2, S, D), jnp.float32),
            pltpu.VMEM((2, S), jnp.int32),
            pltpu.VMEM((2, S), jnp.int32),
            pltpu.VMEM((2, S), jnp.float32),
            pltpu.VMEM((2, S), jnp.float32),
            pltpu.VMEM((2 * S,), jnp.float32),
            pltpu.VMEM((2 * S,), jnp.float32),
            pltpu.VMEM((_L,), jnp.float32),
            pltpu.VMEM((_EPW,), jnp.float32),
            [[pltpu.SemaphoreType.DMA] * 4] * 2,
        ],
    )
    def sc_kernel(edge_hbm, pos_hbm, field_hbm, unc_hbm, emat_hbm,
                  mid0_hbm, mid1_hbm, out_hbm,
                  edge_v, sd_idx, rows_pos, rows_field,
                  mid0_v, mid1_v, g0rows, g1rows,
                  em0, em1, m0v, m1v, logit_v, dist_v, uncv, out_acc,
                  sems):
        wid = lax.axis_index("s") * _NC + lax.axis_index("c")
        base = wid * _EPW

        iota = jnp.arange(_L, dtype=jnp.int32)
        row_idx = [g * _L + iota for g in range(8)]
        zeros16 = jnp.zeros((_L,), jnp.float32)

        pltpu.sync_copy(unc_hbm, uncv)
        pltpu.sync_copy(edge_hbm.at[pl.ds(2 * base, 2 * _EPW)], edge_v)
        for h in range(H):
            pltpu.sync_copy(mid0_hbm.at[h, pl.ds(base, _EPW), :], mid0_v.at[h])
            pltpu.sync_copy(mid1_hbm.at[h, pl.ds(base, _EPW), :], mid1_v.at[h])

        for half in range(2):
            for g in range(2):
                v = plsc.load_gather(edge_v, [(g * _L + iota) * 2 + half])
                sd_idx[pl.ds(half * _EPW + g * _L, _L)] = v

        cps = []
        for h in range(H):
            cps.append(pltpu.async_copy(pos_hbm.at[h].at[sd_idx],
                                        rows_pos.at[h], sems[0][0]))
            cps.append(pltpu.async_copy(field_hbm.at[h].at[sd_idx],
                                        rows_field.at[h], sems[0][1]))
        for cp in cps:
            cp.wait()

        out_acc[pl.ds(0, _L)] = zeros16
        out_acc[pl.ds(_L, _L)] = zeros16
        u_vec = uncv[...]

        def issue(p, b):
            h = lax.shift_right_logical(p, 5)
            jj = lax.bitwise_and(p, _EPW - 1)
            h_b = _i32(h)
            jj_b = _i32(jj)
            pltpu.async_copy(pos_hbm.at[h].at[mid0_v.at[h, jj]],
                             g0rows.at[b], sems[b][0])
            pltpu.async_copy(pos_hbm.at[h].at[mid1_v.at[h, jj]],
                             g1rows.at[b], sems[b][1])
            src_b = plsc.load_gather(edge_v, [jj_b * 2])
            dst_b = plsc.load_gather(edge_v, [jj_b * 2 + 1])
            bb = _i32(b)
            for g in range(8):
                mv0 = plsc.load_gather(mid0_v, [h_b, jj_b, row_idx[g]])
                mv1 = plsc.load_gather(mid1_v, [h_b, jj_b, row_idx[g]])
                plsc.store_scatter(em0, [bb, row_idx[g]],
                                   mv0 * N_NODES + dst_b)
                plsc.store_scatter(em1, [bb, row_idx[g]],
                                   src_b * N_NODES + mv1)
            pltpu.async_copy(emat_hbm.at[em0.at[b]], m0v.at[b], sems[b][2])
            pltpu.async_copy(emat_hbm.at[em1.at[b]], m1v.at[b], sems[b][3])

        def wait_pair(b):
            pltpu.make_async_copy(pos_hbm.at[0].at[mid0_v.at[0, 0]],
                                  g0rows.at[b], sems[b][0]).wait()
            pltpu.make_async_copy(pos_hbm.at[0].at[mid0_v.at[0, 0]],
                                  g1rows.at[b], sems[b][1]).wait()
            pltpu.make_async_copy(emat_hbm.at[em0.at[b]],
                                  m0v.at[b], sems[b][2]).wait()
            pltpu.make_async_copy(emat_hbm.at[em0.at[b]],
                                  m1v.at[b], sems[b][3]).wait()

        def compute(p, b):
            h = lax.shift_right_logical(p, 5)
            jj = lax.bitwise_and(p, _EPW - 1)
            h_b = _i32(h)
            bb = _i32(b)

            for half in range(2):
                grows = g0rows if half == 0 else g1rows
                p_row = _i32(jj + (half * _EPW))
                f_row = _i32(jj + ((1 - half) * _EPW))

                def d_body(d, accs, grows=grows, p_row=p_row, f_row=f_row):
                    dcol = lax.bitwise_and(d + iota, jnp.int32(D - 1))
                    ps = plsc.load_gather(rows_pos, [h_b, p_row, dcol])
                    fd = plsc.load_gather(rows_field, [h_b, f_row, dcol])
                    out = []
                    for g in range(8):
                        v = plsc.load_gather(grows, [bb, row_idx[g], dcol])
                        df = ps - v
                        out.append(accs[2 * g] + df * fd)
                        out.append(accs[2 * g + 1] + df * df)
                    return tuple(out)

                accs = plsc.parallel_loop(
                    0, D, unroll=4,
                    carry=tuple(zeros16 for _ in range(16)))(d_body)

                mref = m0v if half == 0 else m1v
                for g in range(8):
                    al, ad = accs[2 * g], accs[2 * g + 1]
                    mem = plsc.load_gather(mref, [bb, row_idx[g]])
                    off = half * S + g * _L
                    logit_v[pl.ds(off, _L)] = al + u_vec * mem
                    x = jnp.maximum(ad, jnp.float32(1e-30))
                    dist_v[pl.ds(off, _L)] = ad * _rsqrt(x)

            dmin = dist_v[pl.ds(0, _L)]
            for k in range(1, 16):
                dmin = jnp.minimum(dmin, dist_v[pl.ds(k * _L, _L)])
            m = jnp.maximum(1.0 - jnp.min(dmin, axis=0), jnp.float32(0.0))
            num = zeros16
            den = zeros16
            for k in range(16):
                e = jnp.exp((1.0 - dist_v[pl.ds(k * _L, _L)]) - m)
                num = num + logit_v[pl.ds(k * _L, _L)] * e
                den = den + e
            den = den + jnp.exp(_f32(0.0) - m) * jnp.float32(SENT / _L)
            num_s = _f32(0.0) + jnp.sum(num, axis=0)
            den_s = _f32(0.0) + jnp.sum(den, axis=0)
            val = num_s / den_s

            jj_b = _i32(jj)
            cur = plsc.load_gather(out_acc, [jj_b])
            plsc.store_scatter(out_acc, [jj_b],
                               cur + val * jnp.float32(1.0 / H),
                               mask=iota == 0)

        issue(jnp.int32(0), 0)

        def pair_body(i, carry):
            p0 = 2 * i
            issue(p0 + 1, 1)
            wait_pair(0)
            compute(p0, 0)

            @pl.when(i < _PAIRS // 2 - 1)
            def _():
                issue(p0 + 2, 0)

            wait_pair(1)
            compute(p0 + 1, 1)
            return carry

        lax.fori_loop(0, _PAIRS // 2, pair_body, jnp.int32(0))

        pltpu.sync_copy(out_acc, out_hbm.at[pl.ds(base, _EPW)])

    return sc_kernel


_SC_KERNEL = _make_sc_kernel()


def kernel(edge, pos, field, uncertainty, edge_mat, mid0, mid1):
    unc16 = jnp.broadcast_to(uncertainty.reshape(1), (_L,)).astype(jnp.float32)
    emat_flat = edge_mat.reshape(N_NODES * N_NODES)
    edge_flat = edge.reshape(2 * NE)
    return _SC_KERNEL(edge_flat, pos, field, unc16, emat_flat, mid0, mid1)
